# baseline (speedup 1.0000x reference)
import jax
import jax.numpy as jnp
from jax import lax
from jax.experimental import pallas as pl
from jax.experimental.pallas import tpu as pltpu


def _one(logits_ref, feat_ref, out_ref, conf_ref, k):
    n = logits_ref.shape[1]
    L = logits_ref[k]
    m = jnp.max(L, axis=0)
    ii = lax.broadcasted_iota(jnp.int32, (n, n), 0)
    t = L - m[None, :]
    preds = jnp.min(jnp.where(t == 0.0, ii, n), axis=0)
    s = jnp.sum(jnp.exp(t), axis=0)
    conf_ref[k, 0] = 1.0 / s

    pp = lax.broadcasted_iota(jnp.int32, (n, n), 1)
    hit = preds[:, None] == pp
    winner = jnp.max(jnp.where(hit, ii, -1), axis=0)
    jj16 = lax.broadcasted_iota(jnp.int16, (n, n), 0)
    one = jnp.bfloat16(1.0)
    zero = jnp.bfloat16(0.0)
    Pt = jnp.where(jj16 == winner[None, :].astype(jnp.int16), one, zero)

    inner = (Pt[:, :-2] + Pt[:, 1:-1] + Pt[:, 2:]) * jnp.bfloat16(1.0 / 3.0)
    Mt = jnp.concatenate([Pt[:, :1], inner, Pt[:, -1:]], axis=1)

    out_ref[k] = lax.dot_general(
        feat_ref[k].astype(jnp.bfloat16), Mt,
        dimension_numbers=(((0,), (0,)), ((), ())),
        preferred_element_type=jnp.float32,
    )


def _body(logits_ref, feat_ref, out_ref, conf_ref):
    for k in range(logits_ref.shape[0]):
        _one(logits_ref, feat_ref, out_ref, conf_ref, k)


def kernel(features, position_logits):
    b, n, d = features.shape
    bb = 4
    recon_t, conf3 = pl.pallas_call(
        _body,
        grid=(b // bb,),
        in_specs=[
            pl.BlockSpec((bb, n, n), lambda i: (i, 0, 0)),
            pl.BlockSpec((bb, n, d), lambda i: (i, 0, 0)),
        ],
        out_specs=[
            pl.BlockSpec((bb, d, n), lambda i: (i, 0, 0)),
            pl.BlockSpec((bb, 1, n), lambda i: (i, 0, 0)),
        ],
        out_shape=[
            jax.ShapeDtypeStruct((b, d, n), jnp.float32),
            jax.ShapeDtypeStruct((b, 1, n), jnp.float32),
        ],
        compiler_params=pltpu.CompilerParams(
            dimension_semantics=("parallel",)),
    )(position_logits, features)
    g = int(round(n ** 0.5))
    return (recon_t.reshape(b, d, g, g), conf3.reshape(b, n))
